# baseline (speedup 1.0000x reference)
import functools

import jax
import jax.numpy as jnp
from jax import lax
from jax.experimental import pallas as pl
from jax.experimental.pallas import tpu as pltpu
from jax.experimental.pallas import tpu_sc as plsc

N_FEATURES = 100
N_QUANTILES = 1000
EMB_DIM = 32
N_EMB = (N_QUANTILES + 2) * N_FEATURES

BATCH = 16384
NW = 32
ROWS_W = BATCH // NW
BB = 128
NBB = ROWS_W // BB
FG = 4
NFG = N_FEATURES // FG
N_CHUNKS = NBB * NFG
CHUNK = FG * BB
LANES = 16
DT = EMB_DIM // 8


def _body(x_hbm, off_hbm, emb_hbm, out_hbm,
          x_v, off_v,
          ilo_v, ihi_v, wl0, wl1, wh0, wh1,
          lo0, lo1, hi0, hi1, ov0, ov1,
          gsem0, gsem1, osem0, osem1):
    wid = lax.axis_index("s") * 2 + lax.axis_index("c")
    th = jnp.float32(1e-06)
    one = jnp.float32(1.0)
    nq = jnp.float32(N_QUANTILES)
    iota = lax.iota(jnp.int32, LANES)

    wl_ = (wl0, wl1)
    wh_ = (wh0, wh1)
    lo_ = (lo0, lo1)
    hi_ = (hi0, hi1)
    ov = (ov0, ov1)
    gsem = (gsem0, gsem1)
    osem = (osem0, osem1)

    pltpu.sync_copy(off_hbm, off_v)

    def stage_block(c):
        @pl.when(lax.rem(c, jnp.int32(NFG)) == 0)
        def _():
            b0 = wid * ROWS_W + (c // jnp.int32(NFG)) * BB
            pltpu.sync_copy(x_hbm.at[pl.ds(b0, BB)], x_v)

    def idx_compute(c, s):
        f0 = lax.rem(c, jnp.int32(NFG)) * FG
        @plsc.parallel_loop(0, CHUNK // LANES, unroll=1)
        def idx_step(i):
            lin = iota + i * LANES
            fl = lin >> 7
            b = lin & (BB - 1)
            f = f0 + fl
            off = plsc.load_gather(off_v, [f])
            xm = plsc.load_gather(x_v, [b, f])
            m = 1 - lax.shift_right_logical(plsc.bitcast(xm, jnp.int32), 31)
            xv = jnp.abs(xm)
            xc = jnp.minimum(jnp.maximum(xv, th), one - th)
            y = xc * nq
            yi = y.astype(jnp.int32)
            yh = (y + one).astype(jnp.int32)
            xl = yi.astype(jnp.float32) / nq
            xh = yh.astype(jnp.float32) / nq
            sl = pl.ds(i * LANES, LANES)
            wl_[s][sl] = (xh - xc) * nq
            wh_[s][sl] = (xc - xl) * nq
            plsc.store_scatter(ilo_v, [lin], (yi + 1) * m + off)
            plsc.store_scatter(ihi_v, [lin], (yh + 1) * m + off)

    def fire_gathers(s):
        pltpu.async_copy(emb_hbm.at[ilo_v], lo_[s], gsem[s])
        pltpu.async_copy(emb_hbm.at[ihi_v], hi_[s], gsem[s])

    def wait_gathers(s):
        pltpu.make_async_copy(emb_hbm.at[ilo_v], lo_[s], gsem[s]).wait()
        pltpu.make_async_copy(emb_hbm.at[ihi_v], hi_[s], gsem[s]).wait()

    def lerp(c, s):
        @plsc.parallel_loop(0, CHUNK // LANES, unroll=2)
        def lerp_group(q):
            lin = iota + q * LANES
            fl = lin >> 7
            bv = lin & (BB - 1)
            wl = wl_[s][pl.ds(q * LANES, LANES)]
            wh = wh_[s][pl.ds(q * LANES, LANES)]
            it4 = lax.shift_right_logical(iota, 2)
            ia3 = iota & 3
            for d in range(EMB_DIM):
                dv = ((it4 + d) & 3) * 8 + ((ia3 + (d >> 2)) & 7)
                lo = plsc.load_gather(lo_[s], [lin, dv])
                hi = plsc.load_gather(hi_[s], [lin, dv])
                plsc.store_scatter(ov[s], [fl, dv, bv], lo * wl + hi * wh)

    def fire_out(c, s):
        f0 = lax.rem(c, jnp.int32(NFG)) * FG
        bb = c // jnp.int32(NFG) + wid * NBB
        for fl in range(FG):
            for t in range(DT):
                pltpu.async_copy(ov[s].at[fl, pl.ds(t * 8, 8)],
                                 out_hbm.at[f0 + fl, t, bb], osem[s])

    def wait_out(c, s):
        f0 = lax.rem(c, jnp.int32(NFG)) * FG
        bb = c // jnp.int32(NFG) + wid * NBB
        for fl in range(FG):
            for t in range(DT):
                pltpu.make_async_copy(ov[s].at[fl, pl.ds(t * 8, 8)],
                                      out_hbm.at[f0 + fl, t, bb],
                                      osem[s]).wait()

    stage_block(jnp.int32(0))
    idx_compute(jnp.int32(0), 0)
    fire_gathers(0)

    def pair_body(p, carry):
        for par in (0, 1):
            c = 2 * p + par
            s = par
            with jax.named_scope("wait_g"):
                wait_gathers(s)

            @pl.when(c < N_CHUNKS - 1)
            def _():
                with jax.named_scope("stage_idx"):
                    stage_block(c + 1)
                    idx_compute(c + 1, 1 - s)

                @pl.when(c >= 1)
                def _():
                    with jax.named_scope("wait_o"):
                        wait_out(c - 1, 1 - s)
                fire_gathers(1 - s)
            with jax.named_scope("lerp"):
                lerp(c, s)
            fire_out(c, s)
        return carry

    lax.fori_loop(0, N_CHUNKS // 2, pair_body, 0)
    wait_out(jnp.int32(N_CHUNKS - 2), 0)
    wait_out(jnp.int32(N_CHUNKS - 1), 1)


@jax.jit
def kernel(x, mask, rand_table, emb):
    table_shift = jnp.int32(N_EMB) * jnp.asarray(rand_table, jnp.int32)
    off = (N_QUANTILES + 2) * jnp.arange(N_FEATURES, dtype=jnp.int32) + table_shift
    off_pad = jnp.zeros((128,), jnp.int32).at[:N_FEATURES].set(off)

    mesh = plsc.VectorSubcoreMesh(core_axis_name="c", subcore_axis_name="s")
    run = pl.kernel(
        _body,
        out_type=jax.ShapeDtypeStruct(
            (N_FEATURES, DT, BATCH // BB, 8, BB), jnp.float32),
        mesh=mesh,
        compiler_params=pltpu.CompilerParams(needs_layout_passes=False,
                                             use_tc_tiling_on_sc=False),
        scratch_types=[
            pltpu.VMEM((BB, N_FEATURES), jnp.float32),
            pltpu.VMEM((128,), jnp.int32),
            pltpu.VMEM((CHUNK,), jnp.int32),
            pltpu.VMEM((CHUNK,), jnp.int32),
            pltpu.VMEM((CHUNK,), jnp.float32),
            pltpu.VMEM((CHUNK,), jnp.float32),
            pltpu.VMEM((CHUNK,), jnp.float32),
            pltpu.VMEM((CHUNK,), jnp.float32),
            pltpu.VMEM((CHUNK, EMB_DIM), jnp.float32),
            pltpu.VMEM((CHUNK, EMB_DIM), jnp.float32),
            pltpu.VMEM((CHUNK, EMB_DIM), jnp.float32),
            pltpu.VMEM((CHUNK, EMB_DIM), jnp.float32),
            pltpu.VMEM((FG, EMB_DIM, BB), jnp.float32),
            pltpu.VMEM((FG, EMB_DIM, BB), jnp.float32),
            pltpu.SemaphoreType.DMA,
            pltpu.SemaphoreType.DMA,
            pltpu.SemaphoreType.DMA,
            pltpu.SemaphoreType.DMA,
        ],
    )
    xm = jnp.where(mask != 0, x, -x)
    out5 = run(xm, off_pad, emb)
    out = out5.transpose(0, 1, 3, 2, 4).reshape(N_FEATURES, EMB_DIM, BATCH)
    return out.transpose(2, 0, 1)


# speedup vs baseline: 1.0522x; 1.0522x over previous
import functools

import jax
import jax.numpy as jnp
from jax import lax
from jax.experimental import pallas as pl
from jax.experimental.pallas import tpu as pltpu
from jax.experimental.pallas import tpu_sc as plsc

N_FEATURES = 100
N_QUANTILES = 1000
EMB_DIM = 32
N_EMB = (N_QUANTILES + 2) * N_FEATURES

BATCH = 16384
NW = 32
ROWS_W = BATCH // NW
BB = 128
NBB = ROWS_W // BB
FG = 4
NFG = N_FEATURES // FG
N_CHUNKS = NBB * NFG
CHUNK = FG * BB
LANES = 16
DT = EMB_DIM // 8


def _body(x_hbm, off_hbm, emb_hbm, out_hbm,
          x_v, off_v,
          ilo_v, ihi_v, wl0, wl1, wh0, wh1,
          lo0, lo1, hi0, hi1, ov0, ov1,
          gsem0, gsem1, osem0, osem1):
    wid = lax.axis_index("s") * 2 + lax.axis_index("c")
    th = jnp.float32(1e-06)
    one = jnp.float32(1.0)
    nq = jnp.float32(N_QUANTILES)
    iota = lax.iota(jnp.int32, LANES)

    wl_ = (wl0, wl1)
    wh_ = (wh0, wh1)
    lo_ = (lo0, lo1)
    hi_ = (hi0, hi1)
    ov = (ov0, ov1)
    gsem = (gsem0, gsem1)
    osem = (osem0, osem1)

    pltpu.sync_copy(off_hbm, off_v)

    def stage_block(c):
        @pl.when(lax.rem(c, jnp.int32(NFG)) == 0)
        def _():
            b0 = wid * ROWS_W + (c // jnp.int32(NFG)) * BB
            pltpu.sync_copy(x_hbm.at[pl.ds(b0, BB)], x_v)

    def idx_compute(c, s):
        f0 = lax.rem(c, jnp.int32(NFG)) * FG
        @plsc.parallel_loop(0, CHUNK // LANES, unroll=1)
        def idx_step(i):
            lin = iota + i * LANES
            fl = lin >> 7
            b = lin & (BB - 1)
            f = f0 + fl
            off = plsc.load_gather(off_v, [f])
            xm = plsc.load_gather(x_v, [b, f])
            m = 1 - lax.shift_right_logical(plsc.bitcast(xm, jnp.int32), 31)
            xv = jnp.abs(xm)
            xc = jnp.minimum(jnp.maximum(xv, th), one - th)
            y = xc * nq
            yi = y.astype(jnp.int32)
            yh = (y + one).astype(jnp.int32)
            xl = yi.astype(jnp.float32) / nq
            xh = yh.astype(jnp.float32) / nq
            sl = pl.ds(i * LANES, LANES)
            wl_[s][sl] = (xh - xc) * nq
            wh_[s][sl] = (xc - xl) * nq
            plsc.store_scatter(ilo_v, [fl, b], (yi + 1) * m + off)
            plsc.store_scatter(ihi_v, [fl, b], (yh + 1) * m + off)

    def fire_gathers(s):
        for j in range(FG):
            pltpu.async_copy(emb_hbm.at[ilo_v.at[j]],
                             lo_[s].at[pl.ds(j * BB, BB)], gsem[s])
            pltpu.async_copy(emb_hbm.at[ihi_v.at[j]],
                             hi_[s].at[pl.ds(j * BB, BB)], gsem[s])

    def wait_gathers(s):
        for j in range(FG):
            pltpu.make_async_copy(emb_hbm.at[ilo_v.at[j]],
                                  lo_[s].at[pl.ds(j * BB, BB)], gsem[s]).wait()
            pltpu.make_async_copy(emb_hbm.at[ihi_v.at[j]],
                                  hi_[s].at[pl.ds(j * BB, BB)], gsem[s]).wait()

    def lerp(c, s):
        @plsc.parallel_loop(0, CHUNK // LANES, unroll=1)
        def lerp_group(q):
            lin = iota + q * LANES
            fl = lin >> 7
            bv = lin & (BB - 1)
            wl = wl_[s][pl.ds(q * LANES, LANES)]
            wh = wh_[s][pl.ds(q * LANES, LANES)]
            it4 = lax.shift_right_logical(iota, 2)
            ia3 = iota & 3
            for d in range(EMB_DIM):
                dv = ((it4 + d) & 3) * 8 + ((ia3 + (d >> 2)) & 7)
                lo = plsc.load_gather(lo_[s], [lin, dv])
                hi = plsc.load_gather(hi_[s], [lin, dv])
                plsc.store_scatter(ov[s], [fl, dv, bv], lo * wl + hi * wh)

    def fire_out(c, s):
        f0 = lax.rem(c, jnp.int32(NFG)) * FG
        bb = c // jnp.int32(NFG) + wid * NBB
        for fl in range(FG):
            for t in range(DT):
                pltpu.async_copy(ov[s].at[fl, pl.ds(t * 8, 8)],
                                 out_hbm.at[f0 + fl, t, bb], osem[s])

    def wait_out(c, s):
        f0 = lax.rem(c, jnp.int32(NFG)) * FG
        bb = c // jnp.int32(NFG) + wid * NBB
        for fl in range(FG):
            for t in range(DT):
                pltpu.make_async_copy(ov[s].at[fl, pl.ds(t * 8, 8)],
                                      out_hbm.at[f0 + fl, t, bb],
                                      osem[s]).wait()

    stage_block(jnp.int32(0))
    idx_compute(jnp.int32(0), 0)
    fire_gathers(0)

    def pair_body(p, carry):
        for par in (0, 1):
            c = 2 * p + par
            s = par
            wait_gathers(s)

            @pl.when(c < N_CHUNKS - 1)
            def _():
                stage_block(c + 1)
                idx_compute(c + 1, 1 - s)

                @pl.when(c >= 1)
                def _():
                    wait_out(c - 1, 1 - s)
                fire_gathers(1 - s)
            lerp(c, s)
            fire_out(c, s)
        return carry

    lax.fori_loop(0, N_CHUNKS // 2, pair_body, 0)
    wait_out(jnp.int32(N_CHUNKS - 2), 0)
    wait_out(jnp.int32(N_CHUNKS - 1), 1)


@jax.jit
def kernel(x, mask, rand_table, emb):
    table_shift = jnp.int32(N_EMB) * jnp.asarray(rand_table, jnp.int32)
    off = (N_QUANTILES + 2) * jnp.arange(N_FEATURES, dtype=jnp.int32) + table_shift
    off_pad = jnp.zeros((128,), jnp.int32).at[:N_FEATURES].set(off)

    mesh = plsc.VectorSubcoreMesh(core_axis_name="c", subcore_axis_name="s")
    run = pl.kernel(
        _body,
        out_type=jax.ShapeDtypeStruct(
            (N_FEATURES, DT, BATCH // BB, 8, BB), jnp.float32),
        mesh=mesh,
        compiler_params=pltpu.CompilerParams(needs_layout_passes=False,
                                             use_tc_tiling_on_sc=False),
        scratch_types=[
            pltpu.VMEM((BB, N_FEATURES), jnp.float32),
            pltpu.VMEM((128,), jnp.int32),
            pltpu.VMEM((FG, BB), jnp.int32),
            pltpu.VMEM((FG, BB), jnp.int32),
            pltpu.VMEM((CHUNK,), jnp.float32),
            pltpu.VMEM((CHUNK,), jnp.float32),
            pltpu.VMEM((CHUNK,), jnp.float32),
            pltpu.VMEM((CHUNK,), jnp.float32),
            pltpu.VMEM((CHUNK, EMB_DIM), jnp.float32),
            pltpu.VMEM((CHUNK, EMB_DIM), jnp.float32),
            pltpu.VMEM((CHUNK, EMB_DIM), jnp.float32),
            pltpu.VMEM((CHUNK, EMB_DIM), jnp.float32),
            pltpu.VMEM((FG, EMB_DIM, BB), jnp.float32),
            pltpu.VMEM((FG, EMB_DIM, BB), jnp.float32),
            pltpu.SemaphoreType.DMA,
            pltpu.SemaphoreType.DMA,
            pltpu.SemaphoreType.DMA,
            pltpu.SemaphoreType.DMA,
        ],
    )
    xm = jnp.where(mask != 0, x, -x)
    out5 = run(xm, off_pad, emb)
    out = out5.transpose(0, 1, 3, 2, 4).reshape(N_FEATURES, EMB_DIM, BATCH)
    return out.transpose(2, 0, 1)
